# baseline (speedup 1.0000x reference)
import functools

import jax
import jax.numpy as jnp
from jax import lax
from jax.experimental import pallas as pl
from jax.experimental.pallas import tpu as pltpu
from jax.experimental.pallas import tpu_sc as plsc

_NUM_EXPERTS = 64
_MULT_MOD = 21
_SEED_MOD = 42
_LANES = 16
_NUM_WORKERS = 32


def _sc_fill(n: int):
    rows = n // _NUM_WORKERS
    steps = 2 * rows // _LANES
    mesh = plsc.VectorSubcoreMesh(core_axis_name="c", subcore_axis_name="s")

    @functools.partial(
        pl.kernel,
        mesh=mesh,
        compiler_params=pltpu.CompilerParams(needs_layout_passes=False),
        out_type=jax.ShapeDtypeStruct((n, 2), jnp.int32),
        scratch_types=[pltpu.VMEM((rows, 2), jnp.int32)],
    )
    def fill(out_hbm, buf):
        i32 = lambda v: jnp.int32(v)
        wid = lax.axis_index("s") * i32(2) + lax.axis_index("c")
        rbase = wid * i32(rows)
        lane = lax.iota(jnp.int32, _LANES)
        cvec = (
            i32(_MULT_MOD) * (lane >> i32(1))
            + i32(_SEED_MOD)
            + (lane & i32(1))
        )
        sbase = i32(_MULT_MOD) * rbase
        row_vec = lane >> i32(1)
        col_vec = lane & i32(1)

        def body(j, carry):
            roff, s = carry
            plsc.store_scatter(
                buf,
                [row_vec + roff, col_vec],
                (cvec + s) & i32(_NUM_EXPERTS - 1),
            )
            return (roff + i32(8), s + i32(_MULT_MOD * 8))

        lax.fori_loop(0, steps, body, (i32(0), sbase))
        pltpu.sync_copy(buf, out_hbm.at[pl.ds(rbase, rows)])

    return fill


def kernel(x):
    batch, seq, _ = x.shape
    n = batch * seq
    return _sc_fill(n)().astype(jnp.int64)


# speedup vs baseline: 1.0004x; 1.0004x over previous
import functools

import jax
import jax.numpy as jnp
from jax import lax
from jax.experimental import pallas as pl
from jax.experimental.pallas import tpu as pltpu
from jax.experimental.pallas import tpu_sc as plsc

_NUM_EXPERTS = 64
_MULT_MOD = 21
_SEED_MOD = 42
_LANES = 16
_NUM_WORKERS = 32


def _sc_fill(n: int):
    rows = n // _NUM_WORKERS
    steps = 2 * rows // _LANES
    mesh = plsc.VectorSubcoreMesh(core_axis_name="c", subcore_axis_name="s")

    @functools.partial(
        pl.kernel,
        mesh=mesh,
        compiler_params=pltpu.CompilerParams(needs_layout_passes=False, use_tc_tiling_on_sc=True),
        out_type=jax.ShapeDtypeStruct((n, 2), jnp.int32),
        scratch_types=[pltpu.VMEM((rows, 2), jnp.int32)],
    )
    def fill(out_hbm, buf):
        i32 = lambda v: jnp.int32(v)
        wid = lax.axis_index("s") * i32(2) + lax.axis_index("c")
        rbase = wid * i32(rows)
        lane = lax.iota(jnp.int32, _LANES)
        cvec = (
            i32(_MULT_MOD) * (lane >> i32(1))
            + i32(_SEED_MOD)
            + (lane & i32(1))
        )
        sbase = i32(_MULT_MOD) * rbase
        row_vec = lane >> i32(1)
        col_vec = lane & i32(1)

        def body(j, carry):
            roff, s = carry
            plsc.store_scatter(
                buf,
                [row_vec + roff, col_vec],
                (cvec + s) & i32(_NUM_EXPERTS - 1),
            )
            return (roff + i32(8), s + i32(_MULT_MOD * 8))

        lax.fori_loop(0, steps, body, (i32(0), sbase))
        pltpu.sync_copy(buf, out_hbm.at[pl.ds(rbase, rows)])

    return fill


def kernel(x):
    batch, seq, _ = x.shape
    n = batch * seq
    return _sc_fill(n)().astype(jnp.int64)


# speedup vs baseline: 14.4183x; 14.4130x over previous
import functools

import jax
import jax.numpy as jnp
from jax import lax
from jax.experimental import pallas as pl
from jax.experimental.pallas import tpu as pltpu
from jax.experimental.pallas import tpu_sc as plsc

_NUM_EXPERTS = 64
_MULT_MOD = 21
_SEED_MOD = 42
_LANES = 16
_NUM_WORKERS = 32


def _sc_fill(n: int):
    n_flat = 2 * n
    chunk = n_flat // _NUM_WORKERS
    steps = chunk // _LANES
    mesh = plsc.VectorSubcoreMesh(core_axis_name="c", subcore_axis_name="s")

    @functools.partial(
        pl.kernel,
        mesh=mesh,
        out_type=jax.ShapeDtypeStruct((n_flat,), jnp.int32),
        scratch_types=[pltpu.VMEM((chunk,), jnp.int32)],
    )
    def fill(out_hbm, buf):
        i32 = lambda v: jnp.int32(v)
        wid = lax.axis_index("s") * i32(2) + lax.axis_index("c")
        base = wid * i32(chunk)
        k = base >> i32(n.bit_length() - 1)
        ibase = base & i32(n - 1)
        lane = lax.iota(jnp.int32, _LANES)
        cvec = i32(_MULT_MOD) * lane
        s0 = i32(_MULT_MOD) * ibase + i32(_SEED_MOD) + k

        def body(j, carry):
            off, s = carry
            buf[pl.ds(off, _LANES)] = (cvec + s) & i32(_NUM_EXPERTS - 1)
            return (off + i32(_LANES), s + i32(_MULT_MOD * _LANES))

        lax.fori_loop(0, steps, body, (i32(0), s0))
        pltpu.sync_copy(buf, out_hbm.at[pl.ds(base, chunk)])

    return fill


def kernel(x):
    batch, seq, _ = x.shape
    n = batch * seq
    out32 = _sc_fill(n)()
    return jnp.swapaxes(out32.reshape(2, n), 0, 1).astype(jnp.int64)
